# baseline (speedup 1.0000x reference)
import functools

import jax
import jax.numpy as jnp
from jax import lax
from jax.experimental import pallas as pl
from jax.experimental.pallas import tpu as pltpu
from jax.experimental.pallas import tpu_sc as plsc

N = 10000
E = 320000
D = 128

NC = 2
NS = 16
NW = NC * NS
EP = E // NW
K = 80
CH = EP // K
NP = 10240
RPT = NP // NS

RR = 4
RI = 8
G = 15


def _spmm_body(x_hbm, row_hbm, col_hbm, w_hbm, out_hbm, accum,
               cb0, cb1, cb2, cb3, cb4, cb5, cb6, cb7,
               rb0, rb1, rb2, rb3, rb4, rb5, rb6, rb7,
               wb0, wb1, wb2, wb3, wb4, wb5, wb6, wb7,
               rows0, rows1, rows2, rows3,
               is0, is1, is2, is3, is4, is5, is6, is7,
               gs0, gs1, gs2, gs3, ss0, ss1, ss2, ss3):
    colb = [cb0, cb1, cb2, cb3, cb4, cb5, cb6, cb7]
    rowb = [rb0, rb1, rb2, rb3, rb4, rb5, rb6, rb7]
    wbuf = [wb0, wb1, wb2, wb3, wb4, wb5, wb6, wb7]
    rows = [rows0, rows1, rows2, rows3]
    isem = [is0, is1, is2, is3, is4, is5, is6, is7]
    gsem = [gs0, gs1, gs2, gs3]
    ssem = [ss0, ss1, ss2, ss3]

    cid = lax.axis_index("c")
    sid = lax.axis_index("s")
    wid = cid * NS + sid
    base = wid * EP

    zero16 = jnp.zeros((16,), jnp.float32)

    def zrow(i, carry):
        for k in range(D // 16):
            rows0[i, pl.ds(k * 16, 16)] = zero16
        return carry

    lax.fori_loop(0, K, zrow, 0)

    def zchunk(i, carry):
        pltpu.sync_copy(rows0, accum.at[pl.ds(sid * RPT + i * K, K)])
        return carry

    lax.fori_loop(0, RPT // K, zchunk, 0)
    plsc.subcore_barrier()

    def idx_issue(i, ch):
        off = pl.multiple_of(base + ch * K, 8)
        pltpu.async_copy(col_hbm.at[pl.ds(off, K)], colb[i], isem[i])
        pltpu.async_copy(row_hbm.at[pl.ds(off, K)], rowb[i], isem[i])
        pltpu.async_copy(w_hbm.at[pl.ds(off, K)], wbuf[i], isem[i])

    def idx_wait(i):
        pltpu.make_async_copy(col_hbm.at[pl.ds(0, K)], colb[i], isem[i]).wait()
        pltpu.make_async_copy(row_hbm.at[pl.ds(0, K)], rowb[i], isem[i]).wait()
        pltpu.make_async_copy(w_hbm.at[pl.ds(0, K)], wbuf[i], isem[i]).wait()

    def gather_issue(r, i):
        pltpu.async_copy(x_hbm.at[colb[i]], rows[r], gsem[r])

    def gather_wait(r):
        pltpu.make_async_copy(x_hbm.at[pl.ds(0, K)], rows[r], gsem[r]).wait()

    def scale(r, i):
        def group(jq, gcarry):
            wtile = wbuf[i][pl.ds(jq * 16, 16)]
            for rr in range(16):
                j = jq * 16 + rr
                wv = lax.gather(
                    wtile, jnp.full((16, 1), rr, jnp.int32),
                    lax.GatherDimensionNumbers(offset_dims=(),
                                               collapsed_slice_dims=(0,),
                                               start_index_map=(0,)),
                    (1,), mode=lax.GatherScatterMode.PROMISE_IN_BOUNDS)
                for k in range(D // 16):
                    sl = pl.ds(k * 16, 16)
                    rows[r][j, sl] = rows[r][j, sl] * wv
            return gcarry

        lax.fori_loop(0, K // 16, group, 0)

    def scatter_issue(r, i):
        pltpu.async_copy(rows[r], accum.at[rowb[i]], ssem[r], add=True)

    def scatter_wait(r):
        pltpu.make_async_copy(rows[r], accum.at[rowb[0]], ssem[r]).wait()

    idx_issue(0, 0)
    idx_issue(1, 1)
    idx_issue(2, 2)
    idx_wait(0)
    gather_issue(0, 0)
    idx_wait(1)
    gather_issue(1, 1)

    def step(b, ch, in_loop):
        rb = b % RR
        r2 = (b + 2) % RR
        i2 = (b + 2) % RI
        i3 = (b + 3) % RI
        gather_wait(rb)
        scale(rb, b)
        scatter_issue(rb, b)
        if in_loop:
            @pl.when(ch >= 2)
            def _():
                scatter_wait(r2)
        elif ch >= 2:
            scatter_wait(r2)
        if in_loop or ch + 3 < CH:
            idx_issue(i3, ch + 3)
        if in_loop or ch + 2 < CH:
            idx_wait(i2)
            gather_issue(r2, i2)

    def outer(g, carry):
        c0 = g * RI
        for b in range(RI):
            step(b, c0 + b, True)
        return carry

    lax.fori_loop(0, G, outer, 0)

    for tch in range(G * RI, CH):
        step(tch % RI, tch, False)
    scatter_wait(3)
    scatter_wait(0)
    plsc.subcore_barrier()

    pltpu.sync_copy(accum.at[pl.ds(sid * RPT, RPT)],
                    out_hbm.at[cid, pl.ds(sid * RPT, RPT)])


_spmm = functools.partial(
    pl.kernel,
    mesh=plsc.VectorSubcoreMesh(core_axis_name="c", subcore_axis_name="s"),
    out_type=jax.ShapeDtypeStruct((NC, NP, D), jnp.float32),
    scratch_types=[
        pltpu.VMEM_SHARED((NP, D), jnp.float32),
    ] + [pltpu.VMEM((K,), jnp.int32) for _ in range(RI)]
      + [pltpu.VMEM((K,), jnp.int32) for _ in range(RI)]
      + [pltpu.VMEM((K,), jnp.float32) for _ in range(RI)]
      + [pltpu.VMEM((K, D), jnp.float32) for _ in range(RR)]
      + [pltpu.SemaphoreType.DMA for _ in range(RI + 2 * RR)],
)(_spmm_body)


BLK = 1000


def _linear_body(p_ref, w_ref, b_ref, a_ref, o_ref):
    s = p_ref[0] + p_ref[1]
    h = lax.dot_general(s, w_ref[...], (((1,), (1,)), ((), ())),
                        preferred_element_type=jnp.float32)
    h = h + b_ref[...]
    o_ref[...] = jnp.where(h >= 0, h, h * a_ref[...])


def _linear_prelu(partials, W, b, alpha):
    return pl.pallas_call(
        _linear_body,
        grid=(N // BLK,),
        in_specs=[
            pl.BlockSpec((NC, BLK, D), lambda i: (0, i, 0)),
            pl.BlockSpec((D, D), lambda i: (0, 0)),
            pl.BlockSpec((1, D), lambda i: (0, 0)),
            pl.BlockSpec((1, D), lambda i: (0, 0)),
        ],
        out_specs=pl.BlockSpec((BLK, D), lambda i: (i, 0)),
        out_shape=jax.ShapeDtypeStruct((N, D), jnp.float32),
    )(partials, W, b.reshape(1, D), alpha.reshape(1, D))


def kernel(x, edge_index, weights, W, b, alpha):
    row = edge_index[0]
    col = edge_index[1]
    partials = _spmm(x, row, col, weights)
    return _linear_prelu(partials, W, b, alpha)
